# baseline (speedup 1.0000x reference)
import jax
import jax.numpy as jnp
from jax import lax
from jax.experimental import pallas as pl
from jax.experimental.pallas import tpu as pltpu
from jax.experimental.pallas import tpu_sc as plsc

N = 10000
E = 320000
F = 128
NCOEF = 8
GP = 12
NTILES = 16
EPT = E // NTILES
CHUNK = 128
NCH = 160
IDXB = 40
NSTG = NCH // IDXB
NPAD = N + 16
RPT = 1000
ROWB = 400
NBLK = N // ROWB


def _deg_body(dstu_hbm, dstd_hbm, degu_hbm, degd_hbm,
              didx_v, ones_v, zb_v, deg_sh, sem):
    c = lax.axis_index("c")
    s = lax.axis_index("s")

    def fill_ones(i, carry):
        ones_v[pl.ds(i * 16, 16)] = jnp.ones((16,), jnp.float32)
        return carry
    lax.fori_loop(0, CHUNK // 16, fill_ones, 0)

    def fill_zeros(i, carry):
        zb_v[pl.ds(i * 16, 16)] = jnp.zeros((16,), jnp.float32)
        return carry
    lax.fori_loop(0, 64, fill_zeros, 0)

    @pl.when(s < 10)
    def _():
        pltpu.sync_copy(zb_v.at[pl.ds(0, 1000)], deg_sh.at[pl.ds(s * 1000, 1000)])
    plsc.subcore_barrier()

    @pl.when(c == 0)
    def _():
        pltpu.sync_copy(dstu_hbm.at[s], didx_v)

    @pl.when(c == 1)
    def _():
        pltpu.sync_copy(dstd_hbm.at[s], didx_v)

    def fire(j):
        pltpu.async_copy(ones_v, deg_sh.at[didx_v.at[j]], sem, add=True)

    def drain(j):
        pltpu.make_async_copy(ones_v, deg_sh.at[didx_v.at[j]], sem).wait()

    for j in range(4):
        fire(j)

    def step(j, carry):
        drain(j)

        @pl.when(j + 4 < NCH)
        def _():
            fire(j + 4)
        return carry
    lax.fori_loop(0, NCH, step, 0)

    plsc.subcore_barrier()

    @pl.when(s < 10)
    def _():
        pltpu.sync_copy(deg_sh.at[pl.ds(s * 1000, 1000)], zb_v.at[pl.ds(0, 1000)])

        @pl.when(c == 0)
        def _():
            pltpu.sync_copy(zb_v.at[pl.ds(0, 1000)], degu_hbm.at[pl.ds(s * 1000, 1000)])

        @pl.when(c == 1)
        def _():
            pltpu.sync_copy(zb_v.at[pl.ds(0, 1000)], degd_hbm.at[pl.ds(s * 1000, 1000)])


def _sct_body(ys_hbm, yi_hbm, srcu_hbm, dstu_hbm, srcd_hbm, dstd_hbm,
              aggs_hbm, aggi_hbm,
              sidx_v, didx_v, ra, rb, agg_sh, gsa, gsb, ssa, ssb):
    c = lax.axis_index("c")
    s = lax.axis_index("s")

    def fill_zeros(i, carry):
        r = i // 8
        k = i - r * 8
        ra[r, pl.ds(k * 16, 16)] = jnp.zeros((16,), jnp.float32)
        return carry
    lax.fori_loop(0, CHUNK * 8, fill_zeros, 0)

    @pl.when(s < 10)
    def _():
        def zero_copy(q, carry):
            pltpu.sync_copy(ra, agg_sh.at[pl.ds(s * RPT + q * CHUNK, CHUNK)])
            return carry
        lax.fori_loop(0, 7, zero_copy, 0)
        pltpu.sync_copy(ra.at[pl.ds(0, 104)], agg_sh.at[pl.ds(s * RPT + 896, 104)])

    plsc.subcore_barrier()

    def conv(y_hbm, src_hbm, dst2_hbm):
        npair = IDXB // 2
        NSUB = 4
        SR = CHUNK // NSUB

        def fire_gather(j, buf, sem):
            for q in range(NSUB):
                pltpu.async_copy(
                    y_hbm.at[sidx_v.at[j, pl.ds(SR * q, SR)]],
                    buf.at[pl.ds(SR * q, SR)], sem)

        def wait_gather(j, buf, sem):
            for q in range(NSUB):
                pltpu.make_async_copy(
                    y_hbm.at[sidx_v.at[j, pl.ds(SR * q, SR)]],
                    buf.at[pl.ds(SR * q, SR)], sem).wait()

        def fire_scatter(j, buf, sem):
            for h in range(2):
                pltpu.async_copy(
                    buf.at[pl.ds(64 * h, 64)],
                    agg_sh.at[didx_v.at[2 * j + h]], sem, add=True)

        def wait_scatter(j, buf, sem):
            for h in range(2):
                pltpu.make_async_copy(
                    buf.at[pl.ds(64 * h, 64)],
                    agg_sh.at[didx_v.at[2 * j + h]], sem).wait()

        def stage(st, carry):
            pltpu.sync_copy(src_hbm.at[s, pl.ds(st * IDXB, IDXB)], sidx_v)
            pltpu.sync_copy(dst2_hbm.at[s, pl.ds(st * IDXB * 2, IDXB * 2)], didx_v)
            fire_gather(0, ra, gsa)

            def pair(p, carry2):
                j = 2 * p
                wait_gather(j, ra, gsa)
                fire_scatter(j, ra, ssa)

                @pl.when(p > 0)
                def _():
                    wait_scatter(j - 1, rb, ssb)
                fire_gather(j + 1, rb, gsb)
                wait_gather(j + 1, rb, gsb)
                fire_scatter(j + 1, rb, ssb)

                @pl.when(p < npair - 1)
                def _():
                    wait_scatter(j, ra, ssa)
                    fire_gather(j + 2, ra, gsa)
                return carry2
            lax.fori_loop(0, npair, pair, 0)
            wait_scatter(IDXB - 2, ra, ssa)
            wait_scatter(IDXB - 1, rb, ssb)
            return carry
        lax.fori_loop(0, NSTG, stage, 0)

    @pl.when(c == 0)
    def _():
        conv(ys_hbm, srcu_hbm, dstu_hbm)

    @pl.when(c == 1)
    def _():
        conv(yi_hbm, srcd_hbm, dstd_hbm)

    plsc.subcore_barrier()

    def copy_out(out_hbm):
        def piece(q, carry):
            off = s * RPT + q * CHUNK
            pltpu.sync_copy(agg_sh.at[pl.ds(off, CHUNK)], ra)
            pltpu.sync_copy(ra, out_hbm.at[pl.ds(off, CHUNK)])
            return carry
        lax.fori_loop(0, 7, piece, 0)
        off = s * RPT + 896
        pltpu.sync_copy(agg_sh.at[pl.ds(off, 104)], ra.at[pl.ds(0, 104)])
        pltpu.sync_copy(ra.at[pl.ds(0, 104)], out_hbm.at[pl.ds(off, 104)])

    @pl.when(s < 10)
    def _():
        @pl.when(c == 0)
        def _():
            copy_out(aggs_hbm)

        @pl.when(c == 1)
        def _():
            copy_out(aggi_hbm)


def _kan_body(x_ref, gp_ref, w_ref, du_ref, dd_ref, zh_ref, ys_ref, yi_ref):
    x = x_ref[...]
    g = [gp_ref[j] for j in range(GP)]
    b = [((x >= g[j][None, :]) & (x < g[j + 1][None, :])).astype(jnp.float32)
         for j in range(GP - 1)]
    for p in range(1, 4):
        nb = []
        for j in range(GP - 1 - p):
            r1 = (1.0 / (g[j + p] - g[j]))[None, :]
            r2 = (1.0 / (g[j + p + 1] - g[j + 1]))[None, :]
            nb.append((x - g[j][None, :]) * r1 * b[j]
                      + (g[j + p + 1][None, :] - x) * r2 * b[j + 1])
        b = nb
    feat = jnp.concatenate([x * jax.nn.sigmoid(x)] + b, axis=1)
    z = jnp.dot(feat, w_ref[...], preferred_element_type=jnp.float32)
    du = du_ref[...]
    dd = dd_ref[...]
    disu = jnp.where(du > 0.0, lax.rsqrt(du), 0.0)
    disd = jnp.where(dd > 0.0, lax.rsqrt(dd), 0.0)
    zh_ref[...] = z[:, :F]
    ys_ref[...] = z[:, F:2 * F] * disu
    yi_ref[...] = z[:, 2 * F:3 * F] * disd


def _fin_body(zh_ref, as_ref, ai_ref, du_ref, dd_ref, o_ref):
    du = du_ref[...]
    dd = dd_ref[...]
    disu = jnp.where(du > 0.0, lax.rsqrt(du), 0.0)
    disd = jnp.where(dd > 0.0, lax.rsqrt(dd), 0.0)
    o_ref[...] = zh_ref[...] + as_ref[...] * disu + ai_ref[...] * disd


def kernel(xe, Lu, Ld, har_base, har_spline, sol_base, sol_spline,
           irr_base, irr_spline, grid):
    f32 = jnp.float32

    def pad_idx(v, pad_val):
        v2 = v.astype(jnp.int32).reshape(NTILES, EPT)
        v2 = jnp.pad(v2, ((0, 0), (0, NCH * CHUNK - EPT)), constant_values=pad_val)
        return v2.reshape(NTILES, NCH, CHUNK)

    srcu = pad_idx(Lu[0], 0)
    dstu = pad_idx(Lu[1], N)
    srcd = pad_idx(Ld[0], 0)
    dstd = pad_idx(Ld[1], N)
    dstu64 = dstu.reshape(NTILES, NCH * 2, 64)
    dstd64 = dstd.reshape(NTILES, NCH * 2, 64)

    base_cat = jnp.concatenate([har_base, sol_base, irr_base], axis=0)
    spl_cat = jnp.concatenate([har_spline, sol_spline, irr_spline], axis=0)
    w = jnp.concatenate(
        [base_cat.T, jnp.transpose(spl_cat, (2, 1, 0)).reshape(NCOEF * F, 3 * F)],
        axis=0)
    gp = grid.astype(f32).T

    mesh = plsc.VectorSubcoreMesh(core_axis_name="c", subcore_axis_name="s")
    deg_call = pl.kernel(
        _deg_body,
        out_type=(jax.ShapeDtypeStruct((N,), f32),
                  jax.ShapeDtypeStruct((N,), f32)),
        mesh=mesh,
        scratch_types=[
            pltpu.VMEM((NCH, CHUNK), jnp.int32),
            pltpu.VMEM((CHUNK,), f32),
            pltpu.VMEM((1024,), f32),
            pltpu.VMEM_SHARED((NPAD,), f32),
            pltpu.SemaphoreType.DMA,
        ],
    )
    degu, degd = deg_call(dstu, dstd)
    du2 = degu.reshape(N, 1)
    dd2 = degd.reshape(N, 1)

    zh, ys, yi = pl.pallas_call(
        _kan_body,
        grid=(NBLK,),
        in_specs=[
            pl.BlockSpec((ROWB, F), lambda i: (i, 0)),
            pl.BlockSpec((GP, F), lambda i: (0, 0)),
            pl.BlockSpec(((1 + NCOEF) * F, 3 * F), lambda i: (0, 0)),
            pl.BlockSpec((ROWB, 1), lambda i: (i, 0)),
            pl.BlockSpec((ROWB, 1), lambda i: (i, 0)),
        ],
        out_specs=[pl.BlockSpec((ROWB, F), lambda i: (i, 0))] * 3,
        out_shape=[jax.ShapeDtypeStruct((N, F), f32)] * 3,
    )(xe, gp, w, du2, dd2)

    sct_call = pl.kernel(
        _sct_body,
        out_type=(jax.ShapeDtypeStruct((N, F), f32),
                  jax.ShapeDtypeStruct((N, F), f32)),
        mesh=mesh,
        scratch_types=[
            pltpu.VMEM((IDXB, CHUNK), jnp.int32),
            pltpu.VMEM((IDXB * 2, 64), jnp.int32),
            pltpu.VMEM((CHUNK, F), f32),
            pltpu.VMEM((CHUNK, F), f32),
            pltpu.VMEM_SHARED((NPAD, F), f32),
            pltpu.SemaphoreType.DMA,
            pltpu.SemaphoreType.DMA,
            pltpu.SemaphoreType.DMA,
            pltpu.SemaphoreType.DMA,
        ],
        compiler_params=pltpu.CompilerParams(use_tc_tiling_on_sc=False),
    )
    aggs, aggi = sct_call(ys, yi, srcu, dstu64, srcd, dstd64)

    out = pl.pallas_call(
        _fin_body,
        grid=(NBLK,),
        in_specs=[
            pl.BlockSpec((ROWB, F), lambda i: (i, 0)),
            pl.BlockSpec((ROWB, F), lambda i: (i, 0)),
            pl.BlockSpec((ROWB, F), lambda i: (i, 0)),
            pl.BlockSpec((ROWB, 1), lambda i: (i, 0)),
            pl.BlockSpec((ROWB, 1), lambda i: (i, 0)),
        ],
        out_specs=pl.BlockSpec((ROWB, F), lambda i: (i, 0)),
        out_shape=jax.ShapeDtypeStruct((N, F), f32),
    )(zh, aggs, aggi, du2, dd2)
    return out


# speedup vs baseline: 1.1084x; 1.1084x over previous
import jax
import jax.numpy as jnp
from jax import lax
from jax.experimental import pallas as pl
from jax.experimental.pallas import tpu as pltpu
from jax.experimental.pallas import tpu_sc as plsc

N = 10000
E = 320000
F = 128
NCOEF = 8
GP = 12
NTILES = 16
EPT = E // NTILES
CHUNK = 128
NCH = 160
IDXB = 40
NSTG = NCH // IDXB
NPAD = N + 16
RPT = 1000
ROWB = 400
NBLK = N // ROWB


def _deg_body(dstu_hbm, dstd_hbm, degu_hbm, degd_hbm,
              didx_v, ones_v, zb_v, deg_sh, sem):
    c = lax.axis_index("c")
    s = lax.axis_index("s")

    def fill_ones(i, carry):
        ones_v[pl.ds(i * 16, 16)] = jnp.ones((16,), jnp.float32)
        return carry
    lax.fori_loop(0, CHUNK // 16, fill_ones, 0)

    def fill_zeros(i, carry):
        zb_v[pl.ds(i * 16, 16)] = jnp.zeros((16,), jnp.float32)
        return carry
    lax.fori_loop(0, 64, fill_zeros, 0)

    @pl.when(s < 10)
    def _():
        pltpu.sync_copy(zb_v.at[pl.ds(0, 1000)], deg_sh.at[pl.ds(s * 1000, 1000)])
    plsc.subcore_barrier()

    @pl.when(c == 0)
    def _():
        pltpu.sync_copy(dstu_hbm.at[s], didx_v)

    @pl.when(c == 1)
    def _():
        pltpu.sync_copy(dstd_hbm.at[s], didx_v)

    def fire(j):
        pltpu.async_copy(ones_v, deg_sh.at[didx_v.at[j]], sem, add=True)

    def drain(j):
        pltpu.make_async_copy(ones_v, deg_sh.at[didx_v.at[j]], sem).wait()

    for j in range(4):
        fire(j)

    def step(j, carry):
        drain(j)

        @pl.when(j + 4 < NCH)
        def _():
            fire(j + 4)
        return carry
    lax.fori_loop(0, NCH, step, 0)

    plsc.subcore_barrier()

    @pl.when(s < 10)
    def _():
        pltpu.sync_copy(deg_sh.at[pl.ds(s * 1000, 1000)], zb_v.at[pl.ds(0, 1000)])

        @pl.when(c == 0)
        def _():
            pltpu.sync_copy(zb_v.at[pl.ds(0, 1000)], degu_hbm.at[pl.ds(s * 1000, 1000)])

        @pl.when(c == 1)
        def _():
            pltpu.sync_copy(zb_v.at[pl.ds(0, 1000)], degd_hbm.at[pl.ds(s * 1000, 1000)])


def _sct_body(ys_hbm, yi_hbm, srcu_hbm, dstu_hbm, srcd_hbm, dstd_hbm,
              aggs_hbm, aggi_hbm,
              sidx_v, didx_v, ra, rb, agg_sh, gsa, gsb, ssa, ssb):
    c = lax.axis_index("c")
    s = lax.axis_index("s")

    def fill_zeros(i, carry):
        r = i // 8
        k = i - r * 8
        ra[r, pl.ds(k * 16, 16)] = jnp.zeros((16,), jnp.float32)
        return carry
    lax.fori_loop(0, CHUNK * 8, fill_zeros, 0)

    @pl.when(s < 10)
    def _():
        def zero_copy(q, carry):
            pltpu.sync_copy(ra, agg_sh.at[pl.ds(s * RPT + q * CHUNK, CHUNK)])
            return carry
        lax.fori_loop(0, 7, zero_copy, 0)
        pltpu.sync_copy(ra.at[pl.ds(0, 104)], agg_sh.at[pl.ds(s * RPT + 896, 104)])

    plsc.subcore_barrier()

    def conv(y_hbm, src_hbm, dst2_hbm):
        npair = IDXB // 2
        NSUB = 4
        SR = CHUNK // NSUB

        def fire_gather(j, buf, sem):
            for q in range(NSUB):
                pltpu.async_copy(
                    y_hbm.at[sidx_v.at[j, pl.ds(SR * q, SR)]],
                    buf.at[pl.ds(SR * q, SR)], sem)

        def wait_gather(j, buf, sem):
            for q in range(NSUB):
                pltpu.make_async_copy(
                    y_hbm.at[sidx_v.at[j, pl.ds(SR * q, SR)]],
                    buf.at[pl.ds(SR * q, SR)], sem).wait()

        def stage(st, carry):
            pltpu.sync_copy(src_hbm.at[s, pl.ds(st * IDXB, IDXB)], sidx_v)
            pltpu.sync_copy(dst2_hbm.at[s, pl.ds(st * IDXB, IDXB)], didx_v)
            fire_gather(0, ra, gsa)

            def pair(p, carry2):
                j = 2 * p
                fire_gather(j + 1, rb, gsb)
                wait_gather(j, ra, gsa)
                pltpu.sync_copy(ra, agg_sh.at[didx_v.at[j]], add=True)

                @pl.when(p < npair - 1)
                def _():
                    fire_gather(j + 2, ra, gsa)
                wait_gather(j + 1, rb, gsb)
                pltpu.sync_copy(rb, agg_sh.at[didx_v.at[j + 1]], add=True)
                return carry2
            lax.fori_loop(0, npair, pair, 0)
            return carry
        lax.fori_loop(0, NSTG, stage, 0)

    @pl.when(c == 0)
    def _():
        conv(ys_hbm, srcu_hbm, dstu_hbm)

    @pl.when(c == 1)
    def _():
        conv(yi_hbm, srcd_hbm, dstd_hbm)

    plsc.subcore_barrier()

    def copy_out(out_hbm):
        def piece(q, carry):
            off = s * RPT + q * CHUNK
            pltpu.sync_copy(agg_sh.at[pl.ds(off, CHUNK)], ra)
            pltpu.sync_copy(ra, out_hbm.at[pl.ds(off, CHUNK)])
            return carry
        lax.fori_loop(0, 7, piece, 0)
        off = s * RPT + 896
        pltpu.sync_copy(agg_sh.at[pl.ds(off, 104)], ra.at[pl.ds(0, 104)])
        pltpu.sync_copy(ra.at[pl.ds(0, 104)], out_hbm.at[pl.ds(off, 104)])

    @pl.when(s < 10)
    def _():
        @pl.when(c == 0)
        def _():
            copy_out(aggs_hbm)

        @pl.when(c == 1)
        def _():
            copy_out(aggi_hbm)


def _kan_body(x_ref, gp_ref, w_ref, du_ref, dd_ref, zh_ref, ys_ref, yi_ref):
    x = x_ref[...]
    g = [gp_ref[j] for j in range(GP)]
    b = [((x >= g[j][None, :]) & (x < g[j + 1][None, :])).astype(jnp.float32)
         for j in range(GP - 1)]
    for p in range(1, 4):
        nb = []
        for j in range(GP - 1 - p):
            r1 = (1.0 / (g[j + p] - g[j]))[None, :]
            r2 = (1.0 / (g[j + p + 1] - g[j + 1]))[None, :]
            nb.append((x - g[j][None, :]) * r1 * b[j]
                      + (g[j + p + 1][None, :] - x) * r2 * b[j + 1])
        b = nb
    feat = jnp.concatenate([x * jax.nn.sigmoid(x)] + b, axis=1)
    z = jnp.dot(feat, w_ref[...], preferred_element_type=jnp.float32)
    du = du_ref[...]
    dd = dd_ref[...]
    disu = jnp.where(du > 0.0, lax.rsqrt(du), 0.0)
    disd = jnp.where(dd > 0.0, lax.rsqrt(dd), 0.0)
    zh_ref[...] = z[:, :F]
    ys_ref[...] = z[:, F:2 * F] * disu
    yi_ref[...] = z[:, 2 * F:3 * F] * disd


def _fin_body(zh_ref, as_ref, ai_ref, du_ref, dd_ref, o_ref):
    du = du_ref[...]
    dd = dd_ref[...]
    disu = jnp.where(du > 0.0, lax.rsqrt(du), 0.0)
    disd = jnp.where(dd > 0.0, lax.rsqrt(dd), 0.0)
    o_ref[...] = zh_ref[...] + as_ref[...] * disu + ai_ref[...] * disd


def kernel(xe, Lu, Ld, har_base, har_spline, sol_base, sol_spline,
           irr_base, irr_spline, grid):
    f32 = jnp.float32

    def pad_idx(v, pad_val):
        v2 = v.astype(jnp.int32).reshape(NTILES, EPT)
        v2 = jnp.pad(v2, ((0, 0), (0, NCH * CHUNK - EPT)), constant_values=pad_val)
        return v2.reshape(NTILES, NCH, CHUNK)

    srcu = pad_idx(Lu[0], 0)
    dstu = pad_idx(Lu[1], N)
    srcd = pad_idx(Ld[0], 0)
    dstd = pad_idx(Ld[1], N)

    base_cat = jnp.concatenate([har_base, sol_base, irr_base], axis=0)
    spl_cat = jnp.concatenate([har_spline, sol_spline, irr_spline], axis=0)
    w = jnp.concatenate(
        [base_cat.T, jnp.transpose(spl_cat, (2, 1, 0)).reshape(NCOEF * F, 3 * F)],
        axis=0)
    gp = grid.astype(f32).T

    mesh = plsc.VectorSubcoreMesh(core_axis_name="c", subcore_axis_name="s")
    deg_call = pl.kernel(
        _deg_body,
        out_type=(jax.ShapeDtypeStruct((N,), f32),
                  jax.ShapeDtypeStruct((N,), f32)),
        mesh=mesh,
        scratch_types=[
            pltpu.VMEM((NCH, CHUNK), jnp.int32),
            pltpu.VMEM((CHUNK,), f32),
            pltpu.VMEM((1024,), f32),
            pltpu.VMEM_SHARED((NPAD,), f32),
            pltpu.SemaphoreType.DMA,
        ],
    )
    degu, degd = deg_call(dstu, dstd)
    du2 = degu.reshape(N, 1)
    dd2 = degd.reshape(N, 1)

    zh, ys, yi = pl.pallas_call(
        _kan_body,
        grid=(NBLK,),
        in_specs=[
            pl.BlockSpec((ROWB, F), lambda i: (i, 0)),
            pl.BlockSpec((GP, F), lambda i: (0, 0)),
            pl.BlockSpec(((1 + NCOEF) * F, 3 * F), lambda i: (0, 0)),
            pl.BlockSpec((ROWB, 1), lambda i: (i, 0)),
            pl.BlockSpec((ROWB, 1), lambda i: (i, 0)),
        ],
        out_specs=[pl.BlockSpec((ROWB, F), lambda i: (i, 0))] * 3,
        out_shape=[jax.ShapeDtypeStruct((N, F), f32)] * 3,
    )(xe, gp, w, du2, dd2)

    sct_call = pl.kernel(
        _sct_body,
        out_type=(jax.ShapeDtypeStruct((N, F), f32),
                  jax.ShapeDtypeStruct((N, F), f32)),
        mesh=mesh,
        scratch_types=[
            pltpu.VMEM((IDXB, CHUNK), jnp.int32),
            pltpu.VMEM((IDXB, CHUNK), jnp.int32),
            pltpu.VMEM((CHUNK, F), f32),
            pltpu.VMEM((CHUNK, F), f32),
            pltpu.VMEM_SHARED((NPAD, F), f32),
            pltpu.SemaphoreType.DMA,
            pltpu.SemaphoreType.DMA,
            pltpu.SemaphoreType.DMA,
            pltpu.SemaphoreType.DMA,
        ],
        compiler_params=pltpu.CompilerParams(use_tc_tiling_on_sc=False),
    )
    aggs, aggi = sct_call(ys, yi, srcu, dstu, srcd, dstd)

    out = pl.pallas_call(
        _fin_body,
        grid=(NBLK,),
        in_specs=[
            pl.BlockSpec((ROWB, F), lambda i: (i, 0)),
            pl.BlockSpec((ROWB, F), lambda i: (i, 0)),
            pl.BlockSpec((ROWB, F), lambda i: (i, 0)),
            pl.BlockSpec((ROWB, 1), lambda i: (i, 0)),
            pl.BlockSpec((ROWB, 1), lambda i: (i, 0)),
        ],
        out_specs=pl.BlockSpec((ROWB, F), lambda i: (i, 0)),
        out_shape=jax.ShapeDtypeStruct((N, F), f32),
    )(zh, aggs, aggi, du2, dd2)
    return out


# speedup vs baseline: 1.1257x; 1.0156x over previous
import jax
import jax.numpy as jnp
from jax import lax
from jax.experimental import pallas as pl
from jax.experimental.pallas import tpu as pltpu
from jax.experimental.pallas import tpu_sc as plsc

N = 10000
E = 320000
F = 128
NCOEF = 8
GP = 12
NTILES = 16
EPT = E // NTILES
CHUNK = 128
NCH = 160
IDXB = 40
NSTG = NCH // IDXB
NPAD = N + 16
RPT = 1000
ROWB = 400
NBLK = N // ROWB


def _deg_body(dstu_hbm, dstd_hbm, degu_hbm, degd_hbm,
              didx_v, ones_v, zb_v, deg_sh, sem):
    c = lax.axis_index("c")
    s = lax.axis_index("s")

    def fill_ones(i, carry):
        ones_v[pl.ds(i * 16, 16)] = jnp.ones((16,), jnp.float32)
        return carry
    lax.fori_loop(0, CHUNK // 16, fill_ones, 0)

    def fill_zeros(i, carry):
        zb_v[pl.ds(i * 16, 16)] = jnp.zeros((16,), jnp.float32)
        return carry
    lax.fori_loop(0, 64, fill_zeros, 0)

    @pl.when(s < 10)
    def _():
        pltpu.sync_copy(zb_v.at[pl.ds(0, 1000)], deg_sh.at[pl.ds(s * 1000, 1000)])
    plsc.subcore_barrier()

    @pl.when(c == 0)
    def _():
        pltpu.sync_copy(dstu_hbm.at[s], didx_v)

    @pl.when(c == 1)
    def _():
        pltpu.sync_copy(dstd_hbm.at[s], didx_v)

    def fire(j):
        pltpu.async_copy(ones_v, deg_sh.at[didx_v.at[j]], sem, add=True)

    def drain(j):
        pltpu.make_async_copy(ones_v, deg_sh.at[didx_v.at[j]], sem).wait()

    for j in range(4):
        fire(j)

    def step(j, carry):
        drain(j)

        @pl.when(j + 4 < NCH)
        def _():
            fire(j + 4)
        return carry
    lax.fori_loop(0, NCH, step, 0)

    plsc.subcore_barrier()

    @pl.when(s < 10)
    def _():
        pltpu.sync_copy(deg_sh.at[pl.ds(s * 1000, 1000)], zb_v.at[pl.ds(0, 1000)])

        @pl.when(c == 0)
        def _():
            pltpu.sync_copy(zb_v.at[pl.ds(0, 1000)], degu_hbm.at[pl.ds(s * 1000, 1000)])

        @pl.when(c == 1)
        def _():
            pltpu.sync_copy(zb_v.at[pl.ds(0, 1000)], degd_hbm.at[pl.ds(s * 1000, 1000)])


def _sct_body(ys_hbm, yi_hbm, srcu_hbm, dstu_hbm, srcd_hbm, dstd_hbm,
              aggs_hbm, aggi_hbm,
              sidx_v, didx_v, ra, rb, agg_sh, gsa, gsb, ssa, ssb):
    c = lax.axis_index("c")
    s = lax.axis_index("s")

    def fill_zeros(i, carry):
        r = i // 8
        k = i - r * 8
        ra[r, pl.ds(k * 16, 16)] = jnp.zeros((16,), jnp.float32)
        return carry
    lax.fori_loop(0, CHUNK * 8, fill_zeros, 0)

    @pl.when(s < 10)
    def _():
        for q in range(7):
            pltpu.async_copy(ra, agg_sh.at[pl.ds(s * RPT + q * CHUNK, CHUNK)], gsa)
        pltpu.async_copy(ra.at[pl.ds(0, 104)],
                         agg_sh.at[pl.ds(s * RPT + 896, 104)], gsb)
        for q in range(7):
            pltpu.make_async_copy(
                ra, agg_sh.at[pl.ds(s * RPT + q * CHUNK, CHUNK)], gsa).wait()
        pltpu.make_async_copy(ra.at[pl.ds(0, 104)],
                              agg_sh.at[pl.ds(s * RPT + 896, 104)], gsb).wait()

    plsc.subcore_barrier()

    def conv(y_hbm, src_hbm, dst2_hbm):
        npair = IDXB // 2
        NSUB = 4
        SR = CHUNK // NSUB

        def fire_gather(j, buf, sem):
            for q in range(NSUB):
                pltpu.async_copy(
                    y_hbm.at[sidx_v.at[j, pl.ds(SR * q, SR)]],
                    buf.at[pl.ds(SR * q, SR)], sem)

        def wait_gather(j, buf, sem):
            for q in range(NSUB):
                pltpu.make_async_copy(
                    y_hbm.at[sidx_v.at[j, pl.ds(SR * q, SR)]],
                    buf.at[pl.ds(SR * q, SR)], sem).wait()

        def stage(st, carry):
            pltpu.sync_copy(src_hbm.at[s, pl.ds(st * IDXB, IDXB)], sidx_v)
            pltpu.sync_copy(dst2_hbm.at[s, pl.ds(st * IDXB, IDXB)], didx_v)
            fire_gather(0, ra, gsa)

            def pair(p, carry2):
                j = 2 * p
                fire_gather(j + 1, rb, gsb)
                wait_gather(j, ra, gsa)
                pltpu.sync_copy(ra, agg_sh.at[didx_v.at[j]], add=True)

                @pl.when(p < npair - 1)
                def _():
                    fire_gather(j + 2, ra, gsa)
                wait_gather(j + 1, rb, gsb)
                pltpu.sync_copy(rb, agg_sh.at[didx_v.at[j + 1]], add=True)
                return carry2
            lax.fori_loop(0, npair, pair, 0)
            return carry
        lax.fori_loop(0, NSTG, stage, 0)

    @pl.when(c == 0)
    def _():
        conv(ys_hbm, srcu_hbm, dstu_hbm)

    @pl.when(c == 1)
    def _():
        conv(yi_hbm, srcd_hbm, dstd_hbm)

    plsc.subcore_barrier()

    def copy_out(out_hbm):
        bufs = (ra, rb)
        sems = (gsa, gsb)
        pieces = [(s * RPT + q * CHUNK, CHUNK if q < 7 else 104) for q in range(8)]
        for q, (off, nr) in enumerate(pieces):
            buf, sem = bufs[q % 2], sems[q % 2]
            if q >= 2:
                poff, pnr = pieces[q - 2]
                pltpu.make_async_copy(buf.at[pl.ds(0, pnr)],
                                      out_hbm.at[pl.ds(poff, pnr)], sem).wait()
            pltpu.sync_copy(agg_sh.at[pl.ds(off, nr)], buf.at[pl.ds(0, nr)])
            pltpu.async_copy(buf.at[pl.ds(0, nr)], out_hbm.at[pl.ds(off, nr)], sem)
        for q in (6, 7):
            off, nr = pieces[q]
            buf, sem = bufs[q % 2], sems[q % 2]
            pltpu.make_async_copy(buf.at[pl.ds(0, nr)],
                                  out_hbm.at[pl.ds(off, nr)], sem).wait()

    @pl.when(s < 10)
    def _():
        @pl.when(c == 0)
        def _():
            copy_out(aggs_hbm)

        @pl.when(c == 1)
        def _():
            copy_out(aggi_hbm)


def _kan_body(x_ref, gp_ref, w_ref, du_ref, dd_ref, zh_ref, ys_ref, yi_ref):
    x = x_ref[...]
    g = [gp_ref[j] for j in range(GP)]
    xm = [x - g[j][None, :] for j in range(GP)]
    b = [((x >= g[j][None, :]) & (x < g[j + 1][None, :])).astype(jnp.float32)
         for j in range(GP - 1)]
    for p in range(1, 4):
        r = (1.0 / (g[p] - g[0]))[None, :]
        b = [(xm[j] * b[j] - xm[j + p + 1] * b[j + 1]) * r
             for j in range(GP - 1 - p)]
    feat = jnp.concatenate([x * jax.nn.sigmoid(x)] + b, axis=1)
    z = jnp.dot(feat, w_ref[...], preferred_element_type=jnp.float32)
    du = du_ref[...]
    dd = dd_ref[...]
    disu = jnp.where(du > 0.0, lax.rsqrt(du), 0.0)
    disd = jnp.where(dd > 0.0, lax.rsqrt(dd), 0.0)
    zh_ref[...] = z[:, :F]
    ys_ref[...] = z[:, F:2 * F] * disu
    yi_ref[...] = z[:, 2 * F:3 * F] * disd


def _fin_body(zh_ref, as_ref, ai_ref, du_ref, dd_ref, o_ref):
    du = du_ref[...]
    dd = dd_ref[...]
    disu = jnp.where(du > 0.0, lax.rsqrt(du), 0.0)
    disd = jnp.where(dd > 0.0, lax.rsqrt(dd), 0.0)
    o_ref[...] = zh_ref[...] + as_ref[...] * disu + ai_ref[...] * disd


def kernel(xe, Lu, Ld, har_base, har_spline, sol_base, sol_spline,
           irr_base, irr_spline, grid):
    f32 = jnp.float32

    def pad_idx(v, pad_val):
        v2 = v.astype(jnp.int32).reshape(NTILES, EPT)
        v2 = jnp.pad(v2, ((0, 0), (0, NCH * CHUNK - EPT)), constant_values=pad_val)
        return v2.reshape(NTILES, NCH, CHUNK)

    srcu = pad_idx(Lu[0], 0)
    dstu = pad_idx(Lu[1], N)
    srcd = pad_idx(Ld[0], 0)
    dstd = pad_idx(Ld[1], N)

    base_cat = jnp.concatenate([har_base, sol_base, irr_base], axis=0)
    spl_cat = jnp.concatenate([har_spline, sol_spline, irr_spline], axis=0)
    w = jnp.concatenate(
        [base_cat.T, jnp.transpose(spl_cat, (2, 1, 0)).reshape(NCOEF * F, 3 * F)],
        axis=0)
    gp = grid.astype(f32).T

    mesh = plsc.VectorSubcoreMesh(core_axis_name="c", subcore_axis_name="s")
    deg_call = pl.kernel(
        _deg_body,
        out_type=(jax.ShapeDtypeStruct((N,), f32),
                  jax.ShapeDtypeStruct((N,), f32)),
        mesh=mesh,
        scratch_types=[
            pltpu.VMEM((NCH, CHUNK), jnp.int32),
            pltpu.VMEM((CHUNK,), f32),
            pltpu.VMEM((1024,), f32),
            pltpu.VMEM_SHARED((NPAD,), f32),
            pltpu.SemaphoreType.DMA,
        ],
    )
    degu, degd = deg_call(dstu, dstd)
    du2 = degu.reshape(N, 1)
    dd2 = degd.reshape(N, 1)

    zh, ys, yi = pl.pallas_call(
        _kan_body,
        grid=(NBLK,),
        in_specs=[
            pl.BlockSpec((ROWB, F), lambda i: (i, 0)),
            pl.BlockSpec((GP, F), lambda i: (0, 0)),
            pl.BlockSpec(((1 + NCOEF) * F, 3 * F), lambda i: (0, 0)),
            pl.BlockSpec((ROWB, 1), lambda i: (i, 0)),
            pl.BlockSpec((ROWB, 1), lambda i: (i, 0)),
        ],
        out_specs=[pl.BlockSpec((ROWB, F), lambda i: (i, 0))] * 3,
        out_shape=[jax.ShapeDtypeStruct((N, F), f32)] * 3,
    )(xe, gp, w, du2, dd2)

    sct_call = pl.kernel(
        _sct_body,
        out_type=(jax.ShapeDtypeStruct((N, F), f32),
                  jax.ShapeDtypeStruct((N, F), f32)),
        mesh=mesh,
        scratch_types=[
            pltpu.VMEM((IDXB, CHUNK), jnp.int32),
            pltpu.VMEM((IDXB, CHUNK), jnp.int32),
            pltpu.VMEM((CHUNK, F), f32),
            pltpu.VMEM((CHUNK, F), f32),
            pltpu.VMEM_SHARED((NPAD, F), f32),
            pltpu.SemaphoreType.DMA,
            pltpu.SemaphoreType.DMA,
            pltpu.SemaphoreType.DMA,
            pltpu.SemaphoreType.DMA,
        ],
        compiler_params=pltpu.CompilerParams(use_tc_tiling_on_sc=False),
    )
    aggs, aggi = sct_call(ys, yi, srcu, dstu, srcd, dstd)

    out = pl.pallas_call(
        _fin_body,
        grid=(NBLK,),
        in_specs=[
            pl.BlockSpec((ROWB, F), lambda i: (i, 0)),
            pl.BlockSpec((ROWB, F), lambda i: (i, 0)),
            pl.BlockSpec((ROWB, F), lambda i: (i, 0)),
            pl.BlockSpec((ROWB, 1), lambda i: (i, 0)),
            pl.BlockSpec((ROWB, 1), lambda i: (i, 0)),
        ],
        out_specs=pl.BlockSpec((ROWB, F), lambda i: (i, 0)),
        out_shape=jax.ShapeDtypeStruct((N, F), f32),
    )(zh, aggs, aggi, du2, dd2)
    return out


# speedup vs baseline: 1.9949x; 1.7722x over previous
import jax
import jax.numpy as jnp
from jax import lax
from jax.experimental import pallas as pl
from jax.experimental.pallas import tpu as pltpu
from jax.experimental.pallas import tpu_sc as plsc

N = 10000
E = 320000
F = 128
NCOEF = 8
GP = 12
NTILES = 16
EPT = E // NTILES
CHUNK = 125
NCH = EPT // CHUNK
IDXB = 40
NSTG = NCH // IDXB
NSUB = 5
SR = CHUNK // NSUB
RPT = 1000
ROWB = 400
NBLK = N // ROWB


def _deg_body(dstu_hbm, dstd_hbm, degu_hbm, degd_hbm,
              didx_v, ones_v, zb_v, deg_sh, sem):
    c = lax.axis_index("c")
    s = lax.axis_index("s")

    def fill_ones(i, carry):
        ones_v[pl.ds(i * 16, 16)] = jnp.ones((16,), jnp.float32)
        return carry
    lax.fori_loop(0, 8, fill_ones, 0)

    def fill_zeros(i, carry):
        zb_v[pl.ds(i * 16, 16)] = jnp.zeros((16,), jnp.float32)
        return carry
    lax.fori_loop(0, 64, fill_zeros, 0)

    @pl.when(s < 10)
    def _():
        pltpu.sync_copy(zb_v.at[pl.ds(0, 1000)], deg_sh.at[pl.ds(s * 1000, 1000)])
    plsc.subcore_barrier()

    @pl.when(c == 0)
    def _():
        pltpu.sync_copy(dstu_hbm.at[s], didx_v)

    @pl.when(c == 1)
    def _():
        pltpu.sync_copy(dstd_hbm.at[s], didx_v)

    def fire(j):
        pltpu.async_copy(ones_v.at[pl.ds(0, CHUNK)], deg_sh.at[didx_v.at[j]],
                         sem, add=True)

    def drain(j):
        pltpu.make_async_copy(ones_v.at[pl.ds(0, CHUNK)],
                              deg_sh.at[didx_v.at[j]], sem).wait()

    for j in range(4):
        fire(j)

    def step(j, carry):
        drain(j)

        @pl.when(j + 4 < NCH)
        def _():
            fire(j + 4)
        return carry
    lax.fori_loop(0, NCH, step, 0)

    plsc.subcore_barrier()

    @pl.when(s < 10)
    def _():
        pltpu.sync_copy(deg_sh.at[pl.ds(s * 1000, 1000)], zb_v.at[pl.ds(0, 1000)])

        @pl.when(c == 0)
        def _():
            pltpu.sync_copy(zb_v.at[pl.ds(0, 1000)], degu_hbm.at[pl.ds(s * 1000, 1000)])

        @pl.when(c == 1)
        def _():
            pltpu.sync_copy(zb_v.at[pl.ds(0, 1000)], degd_hbm.at[pl.ds(s * 1000, 1000)])


def _sct_body(ys_hbm, yi_hbm, srcu_hbm, dstu_hbm, srcd_hbm, dstd_hbm,
              aggs_hbm, aggi_hbm,
              sidx_v, didx_v, ra, rb, agg_sh, gsa, gsb):
    c = lax.axis_index("c")
    s = lax.axis_index("s")

    def fill_zeros(i, carry):
        r = i // 8
        k = i - r * 8
        ra[r, pl.ds(k * 16, 16)] = jnp.zeros((16,), jnp.float32)
        return carry
    lax.fori_loop(0, CHUNK * 8, fill_zeros, 0)

    pieces = [(q * 120, 120) for q in range(8)] + [(960, 40)]

    @pl.when(s < 10)
    def _():
        for off, nr in pieces:
            pltpu.async_copy(ra.at[pl.ds(0, nr)],
                             agg_sh.at[pl.ds(s * RPT + off, nr)], gsa)
        for off, nr in pieces:
            pltpu.make_async_copy(ra.at[pl.ds(0, nr)],
                                  agg_sh.at[pl.ds(s * RPT + off, nr)], gsa).wait()

    plsc.subcore_barrier()

    def conv(y_hbm, src_hbm, dst_hbm):
        npair = IDXB // 2

        def fire_gather(j, buf, sem):
            for q in range(NSUB):
                pltpu.async_copy(
                    y_hbm.at[sidx_v.at[j, pl.ds(SR * q, SR)]],
                    buf.at[pl.ds(SR * q, SR)], sem)

        def wait_gather(j, buf, sem):
            for q in range(NSUB):
                pltpu.make_async_copy(
                    y_hbm.at[sidx_v.at[j, pl.ds(SR * q, SR)]],
                    buf.at[pl.ds(SR * q, SR)], sem).wait()

        def stage(st, carry):
            pltpu.sync_copy(src_hbm.at[s, pl.ds(st * IDXB, IDXB)], sidx_v)
            pltpu.sync_copy(dst_hbm.at[s, pl.ds(st * IDXB, IDXB)], didx_v)
            fire_gather(0, ra, gsa)

            def pair(p, carry2):
                j = 2 * p
                fire_gather(j + 1, rb, gsb)
                wait_gather(j, ra, gsa)
                pltpu.sync_copy(ra, agg_sh.at[didx_v.at[j]], add=True)

                @pl.when(p < npair - 1)
                def _():
                    fire_gather(j + 2, ra, gsa)
                wait_gather(j + 1, rb, gsb)
                pltpu.sync_copy(rb, agg_sh.at[didx_v.at[j + 1]], add=True)
                return carry2
            lax.fori_loop(0, npair, pair, 0)
            return carry
        lax.fori_loop(0, NSTG, stage, 0)

    @pl.when(c == 0)
    def _():
        conv(ys_hbm, srcu_hbm, dstu_hbm)

    @pl.when(c == 1)
    def _():
        conv(yi_hbm, srcd_hbm, dstd_hbm)

    plsc.subcore_barrier()

    def copy_out(out_hbm):
        bufs = (ra, rb)
        sems = (gsa, gsb)
        for q, (off, nr) in enumerate(pieces):
            buf, sem = bufs[q % 2], sems[q % 2]
            if q >= 2:
                poff, pnr = pieces[q - 2]
                pltpu.make_async_copy(
                    buf.at[pl.ds(0, pnr)],
                    out_hbm.at[pl.ds(s * RPT + poff, pnr)], sem).wait()
            pltpu.sync_copy(agg_sh.at[pl.ds(s * RPT + off, nr)], buf.at[pl.ds(0, nr)])
            pltpu.async_copy(buf.at[pl.ds(0, nr)],
                             out_hbm.at[pl.ds(s * RPT + off, nr)], sem)
        for q in (7, 8):
            off, nr = pieces[q]
            buf, sem = bufs[q % 2], sems[q % 2]
            pltpu.make_async_copy(buf.at[pl.ds(0, nr)],
                                  out_hbm.at[pl.ds(s * RPT + off, nr)], sem).wait()

    @pl.when(s < 10)
    def _():
        @pl.when(c == 0)
        def _():
            copy_out(aggs_hbm)

        @pl.when(c == 1)
        def _():
            copy_out(aggi_hbm)


def _kan_body(x_ref, gp_ref, w_ref, zh_ref, xs_ref, xi_ref):
    x = x_ref[...]
    g = [gp_ref[j] for j in range(GP)]
    xm = [x - g[j][None, :] for j in range(GP)]
    b = [((x >= g[j][None, :]) & (x < g[j + 1][None, :])).astype(jnp.float32)
         for j in range(GP - 1)]
    for p in range(1, 4):
        r = (1.0 / (g[p] - g[0]))[None, :]
        b = [(xm[j] * b[j] - xm[j + p + 1] * b[j + 1]) * r
             for j in range(GP - 1 - p)]
    feat = jnp.concatenate([x * jax.nn.sigmoid(x)] + b, axis=1)
    z = jnp.dot(feat, w_ref[...], preferred_element_type=jnp.float32)
    zh_ref[...] = z[:, :F]
    xs_ref[...] = z[:, F:2 * F]
    xi_ref[...] = z[:, 2 * F:3 * F]


def _scale_body(xs_ref, xi_ref, du_ref, dd_ref, ys_ref, yi_ref):
    du = du_ref[...]
    dd = dd_ref[...]
    disu = jnp.where(du > 0.0, lax.rsqrt(du), 0.0)
    disd = jnp.where(dd > 0.0, lax.rsqrt(dd), 0.0)
    ys_ref[...] = xs_ref[...] * disu
    yi_ref[...] = xi_ref[...] * disd


def _fin_body(zh_ref, as_ref, ai_ref, du_ref, dd_ref, o_ref):
    du = du_ref[...]
    dd = dd_ref[...]
    disu = jnp.where(du > 0.0, lax.rsqrt(du), 0.0)
    disd = jnp.where(dd > 0.0, lax.rsqrt(dd), 0.0)
    o_ref[...] = zh_ref[...] + as_ref[...] * disu + ai_ref[...] * disd


def kernel(xe, Lu, Ld, har_base, har_spline, sol_base, sol_spline,
           irr_base, irr_spline, grid):
    f32 = jnp.float32

    def tile_idx(v):
        return v.astype(jnp.int32).reshape(NTILES, NCH, CHUNK)

    srcu = tile_idx(Lu[0])
    dstu = tile_idx(Lu[1])
    srcd = tile_idx(Ld[0])
    dstd = tile_idx(Ld[1])

    base_cat = jnp.concatenate([har_base, sol_base, irr_base], axis=0)
    spl_cat = jnp.concatenate([har_spline, sol_spline, irr_spline], axis=0)
    w = jnp.concatenate(
        [base_cat.T, jnp.transpose(spl_cat, (2, 1, 0)).reshape(NCOEF * F, 3 * F)],
        axis=0)
    gp = grid.astype(f32).T

    mesh = plsc.VectorSubcoreMesh(core_axis_name="c", subcore_axis_name="s")
    deg_call = pl.kernel(
        _deg_body,
        out_type=(jax.ShapeDtypeStruct((N,), f32),
                  jax.ShapeDtypeStruct((N,), f32)),
        mesh=mesh,
        scratch_types=[
            pltpu.VMEM((NCH, CHUNK), jnp.int32),
            pltpu.VMEM((128,), f32),
            pltpu.VMEM((1024,), f32),
            pltpu.VMEM_SHARED((N,), f32),
            pltpu.SemaphoreType.DMA,
        ],
    )
    degu, degd = deg_call(dstu, dstd)
    du2 = degu.reshape(N, 1)
    dd2 = degd.reshape(N, 1)

    rowspec = pl.BlockSpec((ROWB, F), lambda i: (i, 0))
    colspec = pl.BlockSpec((ROWB, 1), lambda i: (i, 0))

    zh, xs, xi = pl.pallas_call(
        _kan_body,
        grid=(NBLK,),
        in_specs=[
            rowspec,
            pl.BlockSpec((GP, F), lambda i: (0, 0)),
            pl.BlockSpec(((1 + NCOEF) * F, 3 * F), lambda i: (0, 0)),
        ],
        out_specs=[rowspec] * 3,
        out_shape=[jax.ShapeDtypeStruct((N, F), f32)] * 3,
    )(xe, gp, w)

    ys, yi = pl.pallas_call(
        _scale_body,
        grid=(NBLK,),
        in_specs=[rowspec, rowspec, colspec, colspec],
        out_specs=[rowspec] * 2,
        out_shape=[jax.ShapeDtypeStruct((N, F), f32)] * 2,
    )(xs, xi, du2, dd2)

    sct_call = pl.kernel(
        _sct_body,
        out_type=(jax.ShapeDtypeStruct((N, F), f32),
                  jax.ShapeDtypeStruct((N, F), f32)),
        mesh=mesh,
        scratch_types=[
            pltpu.VMEM((IDXB, CHUNK), jnp.int32),
            pltpu.VMEM((IDXB, CHUNK), jnp.int32),
            pltpu.VMEM((CHUNK, F), f32),
            pltpu.VMEM((CHUNK, F), f32),
            pltpu.VMEM_SHARED((N, F), f32),
            pltpu.SemaphoreType.DMA,
            pltpu.SemaphoreType.DMA,
        ],
    )
    aggs, aggi = sct_call(ys, yi, srcu, dstu, srcd, dstd)

    out = pl.pallas_call(
        _fin_body,
        grid=(NBLK,),
        in_specs=[rowspec, rowspec, rowspec, colspec, colspec],
        out_specs=rowspec,
        out_shape=jax.ShapeDtypeStruct((N, F), f32),
    )(zh, aggs, aggi, du2, dd2)
    return out
